# Initial kernel scaffold; baseline (speedup 1.0000x reference)
import jax
import jax.numpy as jnp
from jax.experimental import pallas as pl


def kernel(children_memory, children_hidden, Wf, bf, Wiou, biou):
    raise NotImplementedError("write your pallas kernel here")


# speedup vs baseline: 1.5484x; 1.5484x over previous
import functools

import jax
import jax.numpy as jnp
from jax.experimental import pallas as pl

C = 8
B = 4096
D = 768
BB = 256


def _cell_kernel(cm_ref, ch_ref, wft_ref, bf_ref, wiout_ref, biou_ref,
                 nm_ref, nh_ref):
    h = ch_ref[...]
    hs = jnp.sum(h, axis=0)

    iou = jnp.dot(hs, wiout_ref[...],
                  preferred_element_type=jnp.float32) + biou_ref[...]
    input_gate = jax.nn.sigmoid(iou[:, :D])
    output_gate = jax.nn.sigmoid(iou[:, D:2 * D])
    memory_gate = jnp.tanh(iou[:, 2 * D:])

    h2 = h.reshape(C * BB, D)
    f_logits = jnp.dot(h2, wft_ref[...],
                       preferred_element_type=jnp.float32) + bf_ref[...]
    fmem = jax.nn.sigmoid(f_logits) * cm_ref[...].reshape(C * BB, D)
    fsum = jnp.sum(fmem.reshape(C, BB, D), axis=0)

    nm = input_gate * memory_gate + fsum
    nm_ref[...] = nm
    nh_ref[...] = output_gate * jnp.tanh(nm)


@functools.partial(jax.jit, static_argnames=("interpret",))
def kernel(children_memory, children_hidden, Wf, bf, Wiou, biou,
           interpret=False):
    wft = Wf.T
    wiout = Wiou.T
    bf2 = bf.reshape(1, D)
    biou2 = biou.reshape(1, 3 * D)

    grid = (B // BB,)
    nm, nh = pl.pallas_call(
        _cell_kernel,
        grid=grid,
        in_specs=[
            pl.BlockSpec((C, BB, D), lambda i: (0, i, 0)),
            pl.BlockSpec((C, BB, D), lambda i: (0, i, 0)),
            pl.BlockSpec((D, D), lambda i: (0, 0)),
            pl.BlockSpec((1, D), lambda i: (0, 0)),
            pl.BlockSpec((D, 3 * D), lambda i: (0, 0)),
            pl.BlockSpec((1, 3 * D), lambda i: (0, 0)),
        ],
        out_specs=[
            pl.BlockSpec((BB, D), lambda i: (i, 0)),
            pl.BlockSpec((BB, D), lambda i: (i, 0)),
        ],
        out_shape=[
            jax.ShapeDtypeStruct((B, D), jnp.float32),
            jax.ShapeDtypeStruct((B, D), jnp.float32),
        ],
        interpret=interpret,
    )(children_memory, children_hidden, wft, bf2, wiout, biou2)
    return (nm, nh)
